# Initial kernel scaffold; baseline (speedup 1.0000x reference)
import jax
import jax.numpy as jnp
from jax.experimental import pallas as pl


def kernel(x, edge_index, batch, pocket_features, conv_w0, conv_b0, bn_g0, bn_b0, conv_w1, conv_b1, bn_g1, bn_b1, conv_w2, conv_b2, bn_g2, bn_b2, pm_w1, pm_b1, pm_w2, pm_b2, cls_w1, cls_b1, cls_w2, cls_b2):
    raise NotImplementedError("write your pallas kernel here")


# speedup vs baseline: 7.6182x; 7.6182x over previous
import functools

import jax
import jax.numpy as jnp
from jax import lax
from jax.experimental import pallas as pl
from jax.experimental.pallas import tpu as pltpu
from jax.experimental.pallas import tpu_sc as plsc

_N0 = 10000
_NP = 10240
_E0 = 320000
_D = 128
_B = 64
_PD = 28
_NC = 2
_NS = 16
_CH = 80
_K = 128
_EP = _NC * _NS * _CH * _K
_RPT = _NP // _NS
_EPS = 1e-5

@functools.cache
def _get_mesh():
    return plsc.VectorSubcoreMesh(core_axis_name="c", subcore_axis_name="s",
                                  num_cores=_NC, num_subcores=_NS)


def _deg_body(dst_hbm, out_hbm, dst_v, buf, acc):
    c = lax.axis_index("c")
    s = lax.axis_index("s")
    pltpu.sync_copy(dst_hbm.at[c, s], dst_v)
    zero16 = jnp.zeros((16,), jnp.float32)
    one16 = jnp.ones((16,), jnp.float32)

    def fill(r, _):
        buf[0, r, :] = zero16
        buf[1, r, :] = one16
        return 0

    lax.fori_loop(0, _K, fill, 0)
    for j in range(_RPT // _K):
        pltpu.sync_copy(buf.at[0], acc.at[pl.ds(s * _RPT + j * _K, _K)])
    plsc.subcore_barrier()

    def body(ci, _):
        pltpu.sync_copy(buf.at[1], acc.at[dst_v.at[ci]], add=True)
        return 0

    lax.fori_loop(0, _CH, body, 0)
    plsc.subcore_barrier()
    pltpu.sync_copy(acc.at[pl.ds(s * _RPT, _RPT)],
                    out_hbm.at[c, pl.ds(s * _RPT, _RPT)])


def _sc_deg(dst_r):
    return pl.kernel(
        _deg_body,
        out_type=jax.ShapeDtypeStruct((_NC, _NP, 16), jnp.float32),
        mesh=_get_mesh(),
        scratch_types=[
            pltpu.VMEM((_CH, _K), jnp.int32),
            pltpu.VMEM((2, _K, 16), jnp.float32),
            pltpu.VMEM_SHARED((_NP, 16), jnp.float32),
        ],
    )(dst_r)


def _agg_body(p_hbm, src_hbm, dst_hbm, out_hbm, src_v, dst_v, rows_v,
              acc, sem):
    c = lax.axis_index("c")
    s = lax.axis_index("s")
    pltpu.sync_copy(src_hbm.at[c, s], src_v)
    pltpu.sync_copy(dst_hbm.at[c, s], dst_v)
    zero16 = jnp.zeros((16,), jnp.float32)

    def fill(r, _):
        for j in range(_D // 16):
            rows_v[r, pl.ds(j * 16, 16)] = zero16
        return 0

    lax.fori_loop(0, _K, fill, 0)
    for j in range(_RPT // _K):
        pltpu.sync_copy(rows_v, acc.at[pl.ds(s * _RPT + j * _K, _K)])
    plsc.subcore_barrier()

    def body(ci, _):
        pltpu.async_copy(p_hbm.at[src_v.at[ci]], rows_v, sem).wait()
        pltpu.sync_copy(rows_v, acc.at[dst_v.at[ci]], add=True)
        return 0

    lax.fori_loop(0, _CH, body, 0)
    plsc.subcore_barrier()
    pltpu.sync_copy(acc.at[pl.ds(s * _RPT, _RPT)],
                    out_hbm.at[c, pl.ds(s * _RPT, _RPT)])


def _sc_agg(p, src_r, dst_r):
    return pl.kernel(
        _agg_body,
        out_type=jax.ShapeDtypeStruct((_NC, _NP, _D), jnp.float32),
        mesh=_get_mesh(),
        scratch_types=[
            pltpu.VMEM((_CH, _K), jnp.int32),
            pltpu.VMEM((_CH, _K), jnp.int32),
            pltpu.VMEM((_K, _D), jnp.float32),
            pltpu.VMEM_SHARED((_NP, _D), jnp.float32),
            pltpu.SemaphoreType.DMA,
        ],
    )(p, src_r, dst_r)


def _dinv128(deg_ref):
    d = deg_ref[0][:, 0:1] + deg_ref[1][:, 0:1] + 1.0
    return lax.broadcast_in_dim(lax.rsqrt(d), (_NP, _D), (0, 1))


def _row_mask():
    rows = lax.broadcasted_iota(jnp.int32, (_NP, _D), 0)
    return (rows < _N0).astype(jnp.float32)


def _mm_body(x_ref, w_ref, o_ref):
    o_ref[...] = jnp.dot(x_ref[...], w_ref[...],
                         preferred_element_type=jnp.float32)


def _tc_mm(x, w):
    return pl.pallas_call(
        _mm_body, out_shape=jax.ShapeDtypeStruct((_NP, _D), jnp.float32))(x, w)


def _prescale_body(h_ref, deg_ref, o_ref):
    o_ref[...] = _dinv128(deg_ref) * h_ref[...]


def _tc_prescale(h, deg2):
    return pl.pallas_call(
        _prescale_body,
        out_shape=jax.ShapeDtypeStruct((_NP, _D), jnp.float32))(h, deg2)


def _bn_relu(a_ref, p_ref, deg_ref, g_ref, be_ref):
    dinv = _dinv128(deg_ref)
    z = dinv * (a_ref[0] + a_ref[1] + p_ref[...])
    mask = _row_mask()
    m = jnp.sum(z * mask, axis=0, keepdims=True) * (1.0 / _N0)
    d = (z - m) * mask
    var = jnp.sum(d * d, axis=0, keepdims=True) * (1.0 / _N0)
    zn = (z - m) * lax.rsqrt(var + _EPS) * g_ref[...] + be_ref[...]
    return jnp.maximum(zn, 0.0) * mask, dinv, mask


def _layer_body(a_ref, p_ref, deg_ref, g_ref, be_ref, w_ref, o_ref):
    r, dinv, _ = _bn_relu(a_ref, p_ref, deg_ref, g_ref, be_ref)
    o_ref[...] = dinv * jnp.dot(r, w_ref[...],
                                preferred_element_type=jnp.float32)


def _tc_layer(a, p, deg2, g, be, w):
    return pl.pallas_call(
        _layer_body,
        out_shape=jax.ShapeDtypeStruct((_NP, _D), jnp.float32))(
            a, p, deg2, g, be, w)


def _final_body(a_ref, p_ref, deg_ref, g_ref, be_ref, batch_ref, pf_ref,
                pw1_ref, pb1_ref, pw2_ref, pb2_ref,
                cw1_ref, cb1_ref, cw2_ref, cb2_ref, o_ref):
    r, _, mask = _bn_relu(a_ref, p_ref, deg_ref, g_ref, be_ref)
    bids = lax.broadcasted_iota(jnp.int32, (_B, _NP), 0)
    bvals = lax.broadcast_in_dim(batch_ref[...], (_B, _NP), (0, 1))
    oht = (bids == bvals).astype(jnp.float32)
    pooled = jnp.dot(oht, r, preferred_element_type=jnp.float32)
    cnt = jnp.dot(oht, mask, preferred_element_type=jnp.float32)
    mean = pooled / jnp.maximum(cnt, 1.0)
    pe = jnp.maximum(
        jnp.dot(pf_ref[...], pw1_ref[...],
                preferred_element_type=jnp.float32) + pb1_ref[...], 0.0)
    pe2 = jnp.dot(pe, pw2_ref[...],
                  preferred_element_type=jnp.float32) + pb2_ref[...]
    cat = jnp.concatenate(
        [mean, lax.broadcast_in_dim(pe2, (_B, 64), (0, 1))], axis=1)
    l1 = jnp.maximum(
        jnp.dot(cat, cw1_ref[...],
                preferred_element_type=jnp.float32) + cb1_ref[...], 0.0)
    logits = jnp.dot(l1, cw2_ref[...],
                     preferred_element_type=jnp.float32) + cb2_ref[...]
    o_ref[...] = lax.broadcast_in_dim(logits, (_B, _D), (0, 1))


def _tc_final(a, p, deg2, g, be, batch_pad, pf, pw1, pb1, pw2, pb2,
              cw1, cb1, cw2, cb2):
    return pl.pallas_call(
        _final_body,
        out_shape=jax.ShapeDtypeStruct((_B, _D), jnp.float32))(
            a, p, deg2, g, be, batch_pad, pf, pw1, pb1, pw2, pb2,
            cw1, cb1, cw2, cb2)


def kernel(x, edge_index, batch, pocket_features,
           conv_w0, conv_b0, bn_g0, bn_b0,
           conv_w1, conv_b1, bn_g1, bn_b1,
           conv_w2, conv_b2, bn_g2, bn_b2,
           pm_w1, pm_b1, pm_w2, pm_b2,
           cls_w1, cls_b1, cls_w2, cls_b2):
    pad_e = _EP - _E0
    src = jnp.concatenate(
        [edge_index[0], jnp.full((pad_e,), _N0, jnp.int32)])
    dst = jnp.concatenate(
        [edge_index[1], jnp.full((pad_e,), _NP - 1, jnp.int32)])
    src_r = src.reshape(_NC, _NS, _CH, _K)
    dst_r = dst.reshape(_NC, _NS, _CH, _K)
    x_pad = jnp.pad(x, ((0, _NP - _N0), (0, 0)))
    batch_pad = jnp.concatenate(
        [batch, jnp.full((_NP - _N0,), -1, jnp.int32)]).reshape(1, _NP)
    pf = pocket_features.reshape(1, _PD)

    deg2 = _sc_deg(dst_r)
    h0 = _tc_mm(x_pad, conv_w0)
    p0 = _tc_prescale(h0, deg2)
    a0 = _sc_agg(p0, src_r, dst_r)
    p1 = _tc_layer(a0, p0, deg2, bn_g0, bn_b0, conv_w1)
    a1 = _sc_agg(p1, src_r, dst_r)
    p2 = _tc_layer(a1, p1, deg2, bn_g1, bn_b1, conv_w2)
    a2 = _sc_agg(p2, src_r, dst_r)
    out = _tc_final(a2, p2, deg2, bn_g2, bn_b2, batch_pad, pf,
                    pm_w1, pm_b1, pm_w2, pm_b2,
                    cls_w1, cls_b1, cls_w2, cls_b2)
    return out[:, 0]


# speedup vs baseline: 8.6836x; 1.1398x over previous
import functools

import jax
import jax.numpy as jnp
from jax import lax
from jax.experimental import pallas as pl
from jax.experimental.pallas import tpu as pltpu
from jax.experimental.pallas import tpu_sc as plsc

_N0 = 10000
_NP = 10240
_E0 = 320000
_D = 128
_B = 64
_PD = 28
_NC = 2
_NS = 16
_CH = 80
_K = 128
_EP = _NC * _NS * _CH * _K
_RPT = _NP // _NS
_EPS = 1e-5

@functools.cache
def _get_mesh():
    return plsc.VectorSubcoreMesh(core_axis_name="c", subcore_axis_name="s",
                                  num_cores=_NC, num_subcores=_NS)


def _deg_body(dst_hbm, out_hbm, dst_v, buf, acc):
    c = lax.axis_index("c")
    s = lax.axis_index("s")
    pltpu.sync_copy(dst_hbm.at[c, s], dst_v)
    zero16 = jnp.zeros((16,), jnp.float32)
    one16 = jnp.ones((16,), jnp.float32)

    def fill(r, _):
        buf[0, r, :] = zero16
        buf[1, r, :] = one16
        return 0

    lax.fori_loop(0, _K, fill, 0)
    for j in range(_RPT // _K):
        pltpu.sync_copy(buf.at[0], acc.at[pl.ds(s * _RPT + j * _K, _K)])
    plsc.subcore_barrier()

    def body(ci, _):
        pltpu.sync_copy(buf.at[1], acc.at[dst_v.at[ci]], add=True)
        return 0

    lax.fori_loop(0, _CH, body, 0)
    plsc.subcore_barrier()
    pltpu.sync_copy(acc.at[pl.ds(s * _RPT, _RPT)],
                    out_hbm.at[c, pl.ds(s * _RPT, _RPT)])


def _sc_deg(dst_r):
    return pl.kernel(
        _deg_body,
        out_type=jax.ShapeDtypeStruct((_NC, _NP, 16), jnp.float32),
        mesh=_get_mesh(),
        scratch_types=[
            pltpu.VMEM((_CH, _K), jnp.int32),
            pltpu.VMEM((2, _K, 16), jnp.float32),
            pltpu.VMEM_SHARED((_NP, 16), jnp.float32),
        ],
    )(dst_r)


_HCH = _CH // 2


def _agg_body(p_hbm, src_hbm, dst_hbm, out_hbm, src_v, dst_v, rows_v,
              acc, sem0, sem1):
    c = lax.axis_index("c")
    s = lax.axis_index("s")
    zero16 = jnp.zeros((16,), jnp.float32)

    def fill(r, _):
        for j in range(_D // 16):
            rows_v[0, r, pl.ds(j * 16, 16)] = zero16
        return 0

    lax.fori_loop(0, _K, fill, 0)
    for j in range(_RPT // _K):
        pltpu.sync_copy(rows_v.at[0], acc.at[pl.ds(s * _RPT + j * _K, _K)])
    plsc.subcore_barrier()

    for half in range(2):
        pltpu.sync_copy(src_hbm.at[c, s, pl.ds(half * _HCH, _HCH)], src_v)
        pltpu.sync_copy(dst_hbm.at[c, s, pl.ds(half * _HCH, _HCH)], dst_v)
        pltpu.async_copy(p_hbm.at[src_v.at[0]], rows_v.at[0], sem0)

        def body(i, _):
            ci = 2 * i
            pltpu.async_copy(p_hbm.at[src_v.at[ci + 1]], rows_v.at[1], sem1)
            pltpu.make_async_copy(p_hbm.at[src_v.at[ci]], rows_v.at[0],
                                  sem0).wait()
            pltpu.sync_copy(rows_v.at[0], acc.at[dst_v.at[ci]], add=True)

            @pl.when(ci + 2 < _HCH)
            def _():
                pltpu.async_copy(p_hbm.at[src_v.at[ci + 2]], rows_v.at[0],
                                 sem0)

            pltpu.make_async_copy(p_hbm.at[src_v.at[ci + 1]], rows_v.at[1],
                                  sem1).wait()
            pltpu.sync_copy(rows_v.at[1], acc.at[dst_v.at[ci + 1]], add=True)
            return 0

        lax.fori_loop(0, _HCH // 2, body, 0)
    plsc.subcore_barrier()
    pltpu.sync_copy(acc.at[pl.ds(s * _RPT, _RPT)],
                    out_hbm.at[c, pl.ds(s * _RPT, _RPT)])


def _sc_agg(p, src_r, dst_r):
    return pl.kernel(
        _agg_body,
        out_type=jax.ShapeDtypeStruct((_NC, _NP, _D), jnp.float32),
        mesh=_get_mesh(),
        scratch_types=[
            pltpu.VMEM((_HCH, _K), jnp.int32),
            pltpu.VMEM((_HCH, _K), jnp.int32),
            pltpu.VMEM((2, _K, _D), jnp.float32),
            pltpu.VMEM_SHARED((_NP, _D), jnp.float32),
            pltpu.SemaphoreType.DMA,
            pltpu.SemaphoreType.DMA,
        ],
    )(p, src_r, dst_r)


def _dinv128(deg_ref):
    d = deg_ref[0][:, 0:1] + deg_ref[1][:, 0:1] + 1.0
    return lax.broadcast_in_dim(lax.rsqrt(d), (_NP, _D), (0, 1))


def _row_mask():
    rows = lax.broadcasted_iota(jnp.int32, (_NP, _D), 0)
    return (rows < _N0).astype(jnp.float32)


def _mm_body(x_ref, w_ref, o_ref):
    o_ref[...] = jnp.dot(x_ref[...], w_ref[...],
                         preferred_element_type=jnp.float32)


def _tc_mm(x, w):
    return pl.pallas_call(
        _mm_body, out_shape=jax.ShapeDtypeStruct((_NP, _D), jnp.float32))(x, w)


def _prescale_body(h_ref, deg_ref, o_ref):
    o_ref[...] = _dinv128(deg_ref) * h_ref[...]


def _tc_prescale(h, deg2):
    return pl.pallas_call(
        _prescale_body,
        out_shape=jax.ShapeDtypeStruct((_NP, _D), jnp.float32))(h, deg2)


def _bn_relu(a_ref, p_ref, deg_ref, g_ref, be_ref):
    dinv = _dinv128(deg_ref)
    z = dinv * (a_ref[0] + a_ref[1] + p_ref[...])
    mask = _row_mask()
    m = jnp.sum(z * mask, axis=0, keepdims=True) * (1.0 / _N0)
    d = (z - m) * mask
    var = jnp.sum(d * d, axis=0, keepdims=True) * (1.0 / _N0)
    zn = (z - m) * lax.rsqrt(var + _EPS) * g_ref[...] + be_ref[...]
    return jnp.maximum(zn, 0.0) * mask, dinv, mask


def _layer_body(a_ref, p_ref, deg_ref, g_ref, be_ref, w_ref, o_ref):
    r, dinv, _ = _bn_relu(a_ref, p_ref, deg_ref, g_ref, be_ref)
    o_ref[...] = dinv * jnp.dot(r, w_ref[...],
                                preferred_element_type=jnp.float32)


def _tc_layer(a, p, deg2, g, be, w):
    return pl.pallas_call(
        _layer_body,
        out_shape=jax.ShapeDtypeStruct((_NP, _D), jnp.float32))(
            a, p, deg2, g, be, w)


def _final_body(a_ref, p_ref, deg_ref, g_ref, be_ref, batch_ref, pf_ref,
                pw1_ref, pb1_ref, pw2_ref, pb2_ref,
                cw1_ref, cb1_ref, cw2_ref, cb2_ref, o_ref):
    r, _, mask = _bn_relu(a_ref, p_ref, deg_ref, g_ref, be_ref)
    bids = lax.broadcasted_iota(jnp.int32, (_B, _NP), 0)
    bvals = lax.broadcast_in_dim(batch_ref[...], (_B, _NP), (0, 1))
    oht = (bids == bvals).astype(jnp.float32)
    pooled = jnp.dot(oht, r, preferred_element_type=jnp.float32)
    cnt = jnp.dot(oht, mask, preferred_element_type=jnp.float32)
    mean = pooled / jnp.maximum(cnt, 1.0)
    pe = jnp.maximum(
        jnp.dot(pf_ref[...], pw1_ref[...],
                preferred_element_type=jnp.float32) + pb1_ref[...], 0.0)
    pe2 = jnp.dot(pe, pw2_ref[...],
                  preferred_element_type=jnp.float32) + pb2_ref[...]
    cat = jnp.concatenate(
        [mean, lax.broadcast_in_dim(pe2, (_B, 64), (0, 1))], axis=1)
    l1 = jnp.maximum(
        jnp.dot(cat, cw1_ref[...],
                preferred_element_type=jnp.float32) + cb1_ref[...], 0.0)
    logits = jnp.dot(l1, cw2_ref[...],
                     preferred_element_type=jnp.float32) + cb2_ref[...]
    o_ref[...] = lax.broadcast_in_dim(logits, (_B, _D), (0, 1))


def _tc_final(a, p, deg2, g, be, batch_pad, pf, pw1, pb1, pw2, pb2,
              cw1, cb1, cw2, cb2):
    return pl.pallas_call(
        _final_body,
        out_shape=jax.ShapeDtypeStruct((_B, _D), jnp.float32))(
            a, p, deg2, g, be, batch_pad, pf, pw1, pb1, pw2, pb2,
            cw1, cb1, cw2, cb2)


def kernel(x, edge_index, batch, pocket_features,
           conv_w0, conv_b0, bn_g0, bn_b0,
           conv_w1, conv_b1, bn_g1, bn_b1,
           conv_w2, conv_b2, bn_g2, bn_b2,
           pm_w1, pm_b1, pm_w2, pm_b2,
           cls_w1, cls_b1, cls_w2, cls_b2):
    pad_e = _EP - _E0
    src = jnp.concatenate(
        [edge_index[0], jnp.full((pad_e,), _N0, jnp.int32)])
    dst = jnp.concatenate(
        [edge_index[1], jnp.full((pad_e,), _NP - 1, jnp.int32)])
    src_r = src.reshape(_NC, _NS, _CH, _K)
    dst_r = dst.reshape(_NC, _NS, _CH, _K)
    x_pad = jnp.pad(x, ((0, _NP - _N0), (0, 0)))
    batch_pad = jnp.concatenate(
        [batch, jnp.full((_NP - _N0,), -1, jnp.int32)]).reshape(1, _NP)
    pf = pocket_features.reshape(1, _PD)

    deg2 = _sc_deg(dst_r)
    h0 = _tc_mm(x_pad, conv_w0)
    p0 = _tc_prescale(h0, deg2)
    a0 = _sc_agg(p0, src_r, dst_r)
    p1 = _tc_layer(a0, p0, deg2, bn_g0, bn_b0, conv_w1)
    a1 = _sc_agg(p1, src_r, dst_r)
    p2 = _tc_layer(a1, p1, deg2, bn_g1, bn_b1, conv_w2)
    a2 = _sc_agg(p2, src_r, dst_r)
    out = _tc_final(a2, p2, deg2, bn_g2, bn_b2, batch_pad, pf,
                    pm_w1, pm_b1, pm_w2, pm_b2,
                    cls_w1, cls_b1, cls_w2, cls_b2)
    return out[:, 0]


# speedup vs baseline: 8.7618x; 1.0090x over previous
import functools

import jax
import jax.numpy as jnp
from jax import lax
from jax.experimental import pallas as pl
from jax.experimental.pallas import tpu as pltpu
from jax.experimental.pallas import tpu_sc as plsc

_N0 = 10000
_NP = 10240
_E0 = 320000
_D = 128
_B = 64
_PD = 28
_NC = 2
_NS = 16
_CH = 80
_K = 128
_EP = _NC * _NS * _CH * _K
_RPT = _NP // _NS
_EPS = 1e-5

@functools.cache
def _get_mesh():
    return plsc.VectorSubcoreMesh(core_axis_name="c", subcore_axis_name="s",
                                  num_cores=_NC, num_subcores=_NS)


def _deg_body(dst_hbm, out_hbm, dst_v, buf, acc):
    c = lax.axis_index("c")
    s = lax.axis_index("s")
    wid = c * _NS + s
    pltpu.sync_copy(dst_hbm.at[pl.ds(wid * _CH, _CH)], dst_v)
    zero16 = jnp.zeros((16,), jnp.float32)
    one16 = jnp.ones((16,), jnp.float32)

    def fill(r, _):
        buf[0, r, :] = zero16
        buf[1, r, :] = one16
        return 0

    lax.fori_loop(0, _K, fill, 0)
    for j in range(_RPT // _K):
        pltpu.sync_copy(buf.at[0], acc.at[pl.ds(s * _RPT + j * _K, _K)])
    plsc.subcore_barrier()

    def body(ci, _):
        pltpu.sync_copy(buf.at[1], acc.at[dst_v.at[ci]], add=True)
        return 0

    lax.fori_loop(0, _CH, body, 0)
    plsc.subcore_barrier()
    pltpu.sync_copy(acc.at[pl.ds(s * _RPT, _RPT)],
                    out_hbm.at[c, pl.ds(s * _RPT, _RPT)])


def _sc_deg(dst_r):
    return pl.kernel(
        _deg_body,
        out_type=jax.ShapeDtypeStruct((_NC, _NP, 16), jnp.float32),
        mesh=_get_mesh(),
        scratch_types=[
            pltpu.VMEM((_CH, _K), jnp.int32),
            pltpu.VMEM((2, _K, 16), jnp.float32),
            pltpu.VMEM_SHARED((_NP, 16), jnp.float32),
        ],
    )(dst_r)


_SLAB = 40
_CHA = 120
_CHB = _CH * 2 - _CHA


def _agg_pipeline(p_hbm, src_hbm, dst_hbm, src_v, dst_v, rows_v, acc,
                  sem0, sem1, base, nch):
    for h in range(nch // _SLAB):
        off = base + h * _SLAB
        pltpu.sync_copy(src_hbm.at[pl.ds(off, _SLAB)], src_v)
        pltpu.sync_copy(dst_hbm.at[pl.ds(off, _SLAB)], dst_v)
        pltpu.async_copy(p_hbm.at[src_v.at[0]], rows_v.at[0], sem0)

        def body(i, _):
            ci = 2 * i
            pltpu.async_copy(p_hbm.at[src_v.at[ci + 1]], rows_v.at[1], sem1)
            pltpu.make_async_copy(p_hbm.at[src_v.at[ci]], rows_v.at[0],
                                  sem0).wait()
            pltpu.sync_copy(rows_v.at[0], acc.at[dst_v.at[ci]], add=True)

            @pl.when(ci + 2 < _SLAB)
            def _():
                pltpu.async_copy(p_hbm.at[src_v.at[ci + 2]], rows_v.at[0],
                                 sem0)

            pltpu.make_async_copy(p_hbm.at[src_v.at[ci + 1]], rows_v.at[1],
                                  sem1).wait()
            pltpu.sync_copy(rows_v.at[1], acc.at[dst_v.at[ci + 1]], add=True)
            return 0

        lax.fori_loop(0, _SLAB // 2, body, 0)


def _agg_body(p_hbm, src_hbm, dst_hbm, out_hbm, src_v, dst_v, rows_v,
              acc, sem0, sem1):
    c = lax.axis_index("c")
    s = lax.axis_index("s")
    zero16 = jnp.zeros((16,), jnp.float32)

    def fill(r, _):
        for j in range(_D // 16):
            rows_v[0, r, pl.ds(j * 16, 16)] = zero16
        return 0

    lax.fori_loop(0, _K, fill, 0)
    for j in range(_RPT // _K):
        pltpu.sync_copy(rows_v.at[0], acc.at[pl.ds(s * _RPT + j * _K, _K)])
    plsc.subcore_barrier()

    base = (1 - c) * s * _CHA + c * (_NS * _CHA + s * _CHB)

    @pl.when(c == 0)
    def _():
        _agg_pipeline(p_hbm, src_hbm, dst_hbm, src_v, dst_v, rows_v, acc,
                      sem0, sem1, base, _CHA)

    @pl.when(c == 1)
    def _():
        _agg_pipeline(p_hbm, src_hbm, dst_hbm, src_v, dst_v, rows_v, acc,
                      sem0, sem1, base, _CHB)

    plsc.subcore_barrier()
    pltpu.sync_copy(acc.at[pl.ds(s * _RPT, _RPT)],
                    out_hbm.at[c, pl.ds(s * _RPT, _RPT)])


def _sc_agg(p, src_r, dst_r):
    return pl.kernel(
        _agg_body,
        out_type=jax.ShapeDtypeStruct((_NC, _NP, _D), jnp.float32),
        mesh=_get_mesh(),
        scratch_types=[
            pltpu.VMEM((_SLAB, _K), jnp.int32),
            pltpu.VMEM((_SLAB, _K), jnp.int32),
            pltpu.VMEM((2, _K, _D), jnp.float32),
            pltpu.VMEM_SHARED((_NP, _D), jnp.float32),
            pltpu.SemaphoreType.DMA,
            pltpu.SemaphoreType.DMA,
        ],
    )(p, src_r, dst_r)


def _dinv128(deg_ref):
    d = deg_ref[0][:, 0:1] + deg_ref[1][:, 0:1] + 1.0
    return lax.broadcast_in_dim(lax.rsqrt(d), (_NP, _D), (0, 1))


def _row_mask():
    rows = lax.broadcasted_iota(jnp.int32, (_NP, _D), 0)
    return (rows < _N0).astype(jnp.float32)


def _mm_body(x_ref, w_ref, o_ref):
    o_ref[...] = jnp.dot(x_ref[...], w_ref[...],
                         preferred_element_type=jnp.float32)


def _tc_mm(x, w):
    return pl.pallas_call(
        _mm_body, out_shape=jax.ShapeDtypeStruct((_NP, _D), jnp.float32))(x, w)


def _prescale_body(h_ref, deg_ref, o_ref):
    o_ref[...] = _dinv128(deg_ref) * h_ref[...]


def _tc_prescale(h, deg2):
    return pl.pallas_call(
        _prescale_body,
        out_shape=jax.ShapeDtypeStruct((_NP, _D), jnp.float32))(h, deg2)


def _bn_relu(a_ref, p_ref, deg_ref, g_ref, be_ref):
    dinv = _dinv128(deg_ref)
    z = dinv * (a_ref[0] + a_ref[1] + p_ref[...])
    mask = _row_mask()
    m = jnp.sum(z * mask, axis=0, keepdims=True) * (1.0 / _N0)
    d = (z - m) * mask
    var = jnp.sum(d * d, axis=0, keepdims=True) * (1.0 / _N0)
    zn = (z - m) * lax.rsqrt(var + _EPS) * g_ref[...] + be_ref[...]
    return jnp.maximum(zn, 0.0) * mask, dinv, mask


def _layer_body(a_ref, p_ref, deg_ref, g_ref, be_ref, w_ref, o_ref):
    r, dinv, _ = _bn_relu(a_ref, p_ref, deg_ref, g_ref, be_ref)
    o_ref[...] = dinv * jnp.dot(r, w_ref[...],
                                preferred_element_type=jnp.float32)


def _tc_layer(a, p, deg2, g, be, w):
    return pl.pallas_call(
        _layer_body,
        out_shape=jax.ShapeDtypeStruct((_NP, _D), jnp.float32))(
            a, p, deg2, g, be, w)


def _final_body(a_ref, p_ref, deg_ref, g_ref, be_ref, batch_ref, pf_ref,
                pw1_ref, pb1_ref, pw2_ref, pb2_ref,
                cw1_ref, cb1_ref, cw2_ref, cb2_ref, o_ref):
    r, _, mask = _bn_relu(a_ref, p_ref, deg_ref, g_ref, be_ref)
    bids = lax.broadcasted_iota(jnp.int32, (_B, _NP), 0)
    bvals = lax.broadcast_in_dim(batch_ref[...], (_B, _NP), (0, 1))
    oht = (bids == bvals).astype(jnp.float32)
    pooled = jnp.dot(oht, r, preferred_element_type=jnp.float32)
    cnt = jnp.dot(oht, mask, preferred_element_type=jnp.float32)
    mean = pooled / jnp.maximum(cnt, 1.0)
    pe = jnp.maximum(
        jnp.dot(pf_ref[...], pw1_ref[...],
                preferred_element_type=jnp.float32) + pb1_ref[...], 0.0)
    pe2 = jnp.dot(pe, pw2_ref[...],
                  preferred_element_type=jnp.float32) + pb2_ref[...]
    cat = jnp.concatenate(
        [mean, lax.broadcast_in_dim(pe2, (_B, 64), (0, 1))], axis=1)
    l1 = jnp.maximum(
        jnp.dot(cat, cw1_ref[...],
                preferred_element_type=jnp.float32) + cb1_ref[...], 0.0)
    logits = jnp.dot(l1, cw2_ref[...],
                     preferred_element_type=jnp.float32) + cb2_ref[...]
    o_ref[...] = lax.broadcast_in_dim(logits, (_B, _D), (0, 1))


def _tc_final(a, p, deg2, g, be, batch_pad, pf, pw1, pb1, pw2, pb2,
              cw1, cb1, cw2, cb2):
    return pl.pallas_call(
        _final_body,
        out_shape=jax.ShapeDtypeStruct((_B, _D), jnp.float32))(
            a, p, deg2, g, be, batch_pad, pf, pw1, pb1, pw2, pb2,
            cw1, cb1, cw2, cb2)


def kernel(x, edge_index, batch, pocket_features,
           conv_w0, conv_b0, bn_g0, bn_b0,
           conv_w1, conv_b1, bn_g1, bn_b1,
           conv_w2, conv_b2, bn_g2, bn_b2,
           pm_w1, pm_b1, pm_w2, pm_b2,
           cls_w1, cls_b1, cls_w2, cls_b2):
    pad_e = _EP - _E0
    src = jnp.concatenate(
        [edge_index[0], jnp.full((pad_e,), _N0, jnp.int32)])
    dst = jnp.concatenate(
        [edge_index[1], jnp.full((pad_e,), _NP - 1, jnp.int32)])
    src_r = src.reshape(_NC * _NS * _CH, _K)
    dst_r = dst.reshape(_NC * _NS * _CH, _K)
    x_pad = jnp.pad(x, ((0, _NP - _N0), (0, 0)))
    batch_pad = jnp.concatenate(
        [batch, jnp.full((_NP - _N0,), -1, jnp.int32)]).reshape(1, _NP)
    pf = pocket_features.reshape(1, _PD)

    deg2 = _sc_deg(dst_r)
    h0 = _tc_mm(x_pad, conv_w0)
    p0 = _tc_prescale(h0, deg2)
    a0 = _sc_agg(p0, src_r, dst_r)
    p1 = _tc_layer(a0, p0, deg2, bn_g0, bn_b0, conv_w1)
    a1 = _sc_agg(p1, src_r, dst_r)
    p2 = _tc_layer(a1, p1, deg2, bn_g1, bn_b1, conv_w2)
    a2 = _sc_agg(p2, src_r, dst_r)
    out = _tc_final(a2, p2, deg2, bn_g2, bn_b2, batch_pad, pf,
                    pm_w1, pm_b1, pm_w2, pm_b2,
                    cls_w1, cls_b1, cls_w2, cls_b2)
    return out[:, 0]


# speedup vs baseline: 9.2566x; 1.0565x over previous
import functools

import jax
import jax.numpy as jnp
from jax import lax
from jax.experimental import pallas as pl
from jax.experimental.pallas import tpu as pltpu
from jax.experimental.pallas import tpu_sc as plsc

_N0 = 10000
_NP = 10240
_E0 = 320000
_D = 128
_B = 64
_PD = 28
_NC = 2
_NS = 16
_CH = 80
_K = 128
_EP = _NC * _NS * _CH * _K
_RPT = _NP // _NS
_EPS = 1e-5

@functools.cache
def _get_mesh():
    return plsc.VectorSubcoreMesh(core_axis_name="c", subcore_axis_name="s",
                                  num_cores=_NC, num_subcores=_NS)


_DR = _NP // _D
_NW = _NC * _NS


def _deg_body(dst_hbm, out_hbm, dst_v, rows_v, acc):
    c = lax.axis_index("c")
    s = lax.axis_index("s")
    wid = c * _NS + s
    pltpu.sync_copy(dst_hbm.at[pl.ds(wid * _CH, _CH)], dst_v)
    zero16 = jnp.zeros((16,), jnp.float32)
    one16 = jnp.ones((16,), jnp.float32)

    def zfill(r, _):
        for j in range(_D // 16):
            rows_v[r, pl.ds(j * 16, 16)] = zero16
        return 0

    lax.fori_loop(0, _K, zfill, 0)
    for j in range(_RPT // _K):
        pltpu.sync_copy(rows_v, acc.at[pl.ds(s * _RPT + j * _K, _K)])

    def ofill(r, _):
        for j in range(_D // 16):
            rows_v[r, pl.ds(j * 16, 16)] = one16
        return 0

    lax.fori_loop(0, _K, ofill, 0)
    plsc.subcore_barrier()

    def body(ci, _):
        pltpu.sync_copy(rows_v, acc.at[dst_v.at[ci]], add=True)
        return 0

    lax.fori_loop(0, _CH, body, 0)
    plsc.subcore_barrier()
    pltpu.sync_copy(acc.at[pl.ds(s * _RPT, _RPT)],
                    out_hbm.at[c, pl.ds(s * _RPT, _RPT)])


def _sc_deg(dst_r):
    return pl.kernel(
        _deg_body,
        out_type=jax.ShapeDtypeStruct((_NC, _NP, _D), jnp.float32),
        mesh=_get_mesh(),
        scratch_types=[
            pltpu.VMEM((_CH, _K), jnp.int32),
            pltpu.VMEM((_K, _D), jnp.float32),
            pltpu.VMEM_SHARED((_NP, _D), jnp.float32),
        ],
    )(dst_r)


_SLAB = 40
_CHA = 80
_CHB = _CH * 2 - _CHA


def _agg_pipeline(p_hbm, src_hbm, dst_hbm, src_v, dst_v, rows_v, acc,
                  sem0, sem1, base, nch):
    for h in range(nch // _SLAB):
        off = base + h * _SLAB
        pltpu.sync_copy(src_hbm.at[pl.ds(off, _SLAB)], src_v)
        pltpu.sync_copy(dst_hbm.at[pl.ds(off, _SLAB)], dst_v)
        pltpu.async_copy(p_hbm.at[src_v.at[0]], rows_v.at[0], sem0)

        def body(i, _):
            ci = 2 * i
            pltpu.async_copy(p_hbm.at[src_v.at[ci + 1]], rows_v.at[1], sem1)
            pltpu.make_async_copy(p_hbm.at[src_v.at[ci]], rows_v.at[0],
                                  sem0).wait()
            pltpu.sync_copy(rows_v.at[0], acc.at[dst_v.at[ci]], add=True)

            @pl.when(ci + 2 < _SLAB)
            def _():
                pltpu.async_copy(p_hbm.at[src_v.at[ci + 2]], rows_v.at[0],
                                 sem0)

            pltpu.make_async_copy(p_hbm.at[src_v.at[ci + 1]], rows_v.at[1],
                                  sem1).wait()
            pltpu.sync_copy(rows_v.at[1], acc.at[dst_v.at[ci + 1]], add=True)
            return 0

        lax.fori_loop(0, _SLAB // 2, body, 0)


def _agg_body(p_hbm, src_hbm, dst_hbm, out_hbm, src_v, dst_v, rows_v,
              acc, sem0, sem1):
    c = lax.axis_index("c")
    s = lax.axis_index("s")
    zero16 = jnp.zeros((16,), jnp.float32)

    def fill(r, _):
        for j in range(_D // 16):
            rows_v[0, r, pl.ds(j * 16, 16)] = zero16
        return 0

    lax.fori_loop(0, _K, fill, 0)
    for j in range(_RPT // _K):
        pltpu.sync_copy(rows_v.at[0], acc.at[pl.ds(s * _RPT + j * _K, _K)])
    plsc.subcore_barrier()

    base = (1 - c) * s * _CHA + c * (_NS * _CHA + s * _CHB)

    @pl.when(c == 0)
    def _():
        _agg_pipeline(p_hbm, src_hbm, dst_hbm, src_v, dst_v, rows_v, acc,
                      sem0, sem1, base, _CHA)

    @pl.when(c == 1)
    def _():
        _agg_pipeline(p_hbm, src_hbm, dst_hbm, src_v, dst_v, rows_v, acc,
                      sem0, sem1, base, _CHB)

    plsc.subcore_barrier()
    pltpu.sync_copy(acc.at[pl.ds(s * _RPT, _RPT)],
                    out_hbm.at[c, pl.ds(s * _RPT, _RPT)])


def _sc_agg(p, src_r, dst_r):
    return pl.kernel(
        _agg_body,
        out_type=jax.ShapeDtypeStruct((_NC, _NP, _D), jnp.float32),
        mesh=_get_mesh(),
        scratch_types=[
            pltpu.VMEM((_SLAB, _K), jnp.int32),
            pltpu.VMEM((_SLAB, _K), jnp.int32),
            pltpu.VMEM((2, _K, _D), jnp.float32),
            pltpu.VMEM_SHARED((_NP, _D), jnp.float32),
            pltpu.SemaphoreType.DMA,
            pltpu.SemaphoreType.DMA,
        ],
    )(p, src_r, dst_r)


def _degsum_body(d_ref, o_ref):
    o_ref[...] = lax.rsqrt(d_ref[0] + d_ref[1] + 1.0)


def _tc_degsum(deg2):
    return pl.pallas_call(
        _degsum_body,
        out_shape=jax.ShapeDtypeStruct((_NP, _D), jnp.float32))(deg2)


def _dinv128(dinv_ref):
    return dinv_ref[...]


def _row_mask():
    rows = lax.broadcasted_iota(jnp.int32, (_NP, _D), 0)
    return (rows < _N0).astype(jnp.float32)


def _mm_body(x_ref, w_ref, o_ref):
    o_ref[...] = jnp.dot(x_ref[...], w_ref[...],
                         preferred_element_type=jnp.float32)


def _tc_mm(x, w):
    return pl.pallas_call(
        _mm_body, out_shape=jax.ShapeDtypeStruct((_NP, _D), jnp.float32))(x, w)


def _prescale_body(h_ref, deg_ref, o_ref):
    o_ref[...] = _dinv128(deg_ref) * h_ref[...]


def _tc_prescale(h, deg2):
    return pl.pallas_call(
        _prescale_body,
        out_shape=jax.ShapeDtypeStruct((_NP, _D), jnp.float32))(h, deg2)


def _bn_relu(a_ref, p_ref, deg_ref, g_ref, be_ref):
    dinv = _dinv128(deg_ref)
    z = dinv * (a_ref[0] + a_ref[1] + p_ref[...])
    mask = _row_mask()
    m = jnp.sum(z * mask, axis=0, keepdims=True) * (1.0 / _N0)
    d = (z - m) * mask
    var = jnp.sum(d * d, axis=0, keepdims=True) * (1.0 / _N0)
    zn = (z - m) * lax.rsqrt(var + _EPS) * g_ref[...] + be_ref[...]
    return jnp.maximum(zn, 0.0) * mask, dinv, mask


def _layer_body(a_ref, p_ref, deg_ref, g_ref, be_ref, w_ref, o_ref):
    r, dinv, _ = _bn_relu(a_ref, p_ref, deg_ref, g_ref, be_ref)
    o_ref[...] = dinv * jnp.dot(r, w_ref[...],
                                preferred_element_type=jnp.float32)


def _tc_layer(a, p, deg2, g, be, w):
    return pl.pallas_call(
        _layer_body,
        out_shape=jax.ShapeDtypeStruct((_NP, _D), jnp.float32))(
            a, p, deg2, g, be, w)


def _final_body(a_ref, p_ref, deg_ref, g_ref, be_ref, batch_ref, pf_ref,
                pw1_ref, pb1_ref, pw2_ref, pb2_ref,
                cw1_ref, cb1_ref, cw2_ref, cb2_ref, o_ref):
    r, _, mask = _bn_relu(a_ref, p_ref, deg_ref, g_ref, be_ref)
    bids = lax.broadcasted_iota(jnp.int32, (_B, _NP), 0)
    bvals = lax.broadcast_in_dim(batch_ref[...], (_B, _NP), (0, 1))
    oht = (bids == bvals).astype(jnp.float32)
    pooled = jnp.dot(oht, r, preferred_element_type=jnp.float32)
    cnt = jnp.dot(oht, mask, preferred_element_type=jnp.float32)
    mean = pooled / jnp.maximum(cnt, 1.0)
    pe = jnp.maximum(
        jnp.dot(pf_ref[...], pw1_ref[...],
                preferred_element_type=jnp.float32) + pb1_ref[...], 0.0)
    pe2 = jnp.dot(pe, pw2_ref[...],
                  preferred_element_type=jnp.float32) + pb2_ref[...]
    cat = jnp.concatenate(
        [mean, lax.broadcast_in_dim(pe2, (_B, 64), (0, 1))], axis=1)
    l1 = jnp.maximum(
        jnp.dot(cat, cw1_ref[...],
                preferred_element_type=jnp.float32) + cb1_ref[...], 0.0)
    logits = jnp.dot(l1, cw2_ref[...],
                     preferred_element_type=jnp.float32) + cb2_ref[...]
    o_ref[...] = lax.broadcast_in_dim(logits, (_B, _D), (0, 1))


def _tc_final(a, p, deg2, g, be, batch_pad, pf, pw1, pb1, pw2, pb2,
              cw1, cb1, cw2, cb2):
    return pl.pallas_call(
        _final_body,
        out_shape=jax.ShapeDtypeStruct((_B, _D), jnp.float32))(
            a, p, deg2, g, be, batch_pad, pf, pw1, pb1, pw2, pb2,
            cw1, cb1, cw2, cb2)


def kernel(x, edge_index, batch, pocket_features,
           conv_w0, conv_b0, bn_g0, bn_b0,
           conv_w1, conv_b1, bn_g1, bn_b1,
           conv_w2, conv_b2, bn_g2, bn_b2,
           pm_w1, pm_b1, pm_w2, pm_b2,
           cls_w1, cls_b1, cls_w2, cls_b2):
    pad_e = _EP - _E0
    src = jnp.concatenate(
        [edge_index[0], jnp.full((pad_e,), _N0, jnp.int32)])
    dst = jnp.concatenate(
        [edge_index[1], jnp.full((pad_e,), _NP - 1, jnp.int32)])
    src_r = src.reshape(_NC * _NS * _CH, _K)
    dst_r = dst.reshape(_NC * _NS * _CH, _K)
    x_pad = jnp.pad(x, ((0, _NP - _N0), (0, 0)))
    batch_pad = jnp.concatenate(
        [batch, jnp.full((_NP - _N0,), -1, jnp.int32)]).reshape(1, _NP)
    pf = pocket_features.reshape(1, _PD)

    deg2 = _sc_deg(dst_r)
    dinva = _tc_degsum(deg2)
    h0 = _tc_mm(x_pad, conv_w0)
    p0 = _tc_prescale(h0, dinva)
    a0 = _sc_agg(p0, src_r, dst_r)
    p1 = _tc_layer(a0, p0, dinva, bn_g0, bn_b0, conv_w1)
    a1 = _sc_agg(p1, src_r, dst_r)
    p2 = _tc_layer(a1, p1, dinva, bn_g1, bn_b1, conv_w2)
    a2 = _sc_agg(p2, src_r, dst_r)
    out = _tc_final(a2, p2, dinva, bn_g2, bn_b2, batch_pad, pf,
                    pm_w1, pm_b1, pm_w2, pm_b2,
                    cls_w1, cls_b1, cls_w2, cls_b2)
    return out[:, 0]


# speedup vs baseline: 9.5842x; 1.0354x over previous
import functools

import jax
import jax.numpy as jnp
from jax import lax
from jax.experimental import pallas as pl
from jax.experimental.pallas import tpu as pltpu
from jax.experimental.pallas import tpu_sc as plsc

_N0 = 10000
_NP = 10240
_E0 = 320000
_D = 128
_B = 64
_PD = 28
_NC = 2
_NS = 16
_CH = 80
_K = 128
_EP = _NC * _NS * _CH * _K
_RPT = _NP // _NS
_EPS = 1e-5

@functools.cache
def _get_mesh():
    return plsc.VectorSubcoreMesh(core_axis_name="c", subcore_axis_name="s",
                                  num_cores=_NC, num_subcores=_NS)


_DR = _NP // _D
_NW = _NC * _NS


def _deg_body(dst_hbm, out_hbm, dst_v, rows_v, acc):
    c = lax.axis_index("c")
    s = lax.axis_index("s")
    wid = c * _NS + s
    pltpu.sync_copy(dst_hbm.at[pl.ds(wid * _CH, _CH)], dst_v)
    zero16 = jnp.zeros((16,), jnp.float32)
    one16 = jnp.ones((16,), jnp.float32)

    def zfill(r, _):
        for j in range(_D // 16):
            rows_v[r, pl.ds(j * 16, 16)] = zero16
        return 0

    lax.fori_loop(0, _K, zfill, 0)
    for j in range(_RPT // _K):
        pltpu.sync_copy(rows_v, acc.at[pl.ds(s * _RPT + j * _K, _K)])

    def ofill(r, _):
        for j in range(_D // 16):
            rows_v[r, pl.ds(j * 16, 16)] = one16
        return 0

    lax.fori_loop(0, _K, ofill, 0)
    plsc.subcore_barrier()

    def body(ci, _):
        pltpu.sync_copy(rows_v, acc.at[dst_v.at[ci]], add=True)
        return 0

    lax.fori_loop(0, _CH, body, 0)
    plsc.subcore_barrier()
    pltpu.sync_copy(acc.at[pl.ds(s * _RPT, _RPT)],
                    out_hbm.at[c, pl.ds(s * _RPT, _RPT)])


def _sc_deg(dst_r):
    return pl.kernel(
        _deg_body,
        out_type=jax.ShapeDtypeStruct((_NC, _NP, _D), jnp.float32),
        mesh=_get_mesh(),
        scratch_types=[
            pltpu.VMEM((_CH, _K), jnp.int32),
            pltpu.VMEM((_K, _D), jnp.float32),
            pltpu.VMEM_SHARED((_NP, _D), jnp.float32),
        ],
    )(dst_r)


_SLAB = 40
_CHA = 120
_CHB = _CH * 2 - _CHA


def _agg_pipeline(p_hbm, src_hbm, dst_hbm, src_v, dst_v, rows_v, acc,
                  sem0, sem1, base, nch):
    for h in range(nch // _SLAB):
        off = base + h * _SLAB
        pltpu.sync_copy(src_hbm.at[pl.ds(off, _SLAB)], src_v)
        pltpu.sync_copy(dst_hbm.at[pl.ds(off, _SLAB)], dst_v)
        pltpu.async_copy(p_hbm.at[src_v.at[0]], rows_v.at[0], sem0)

        def body(i, _):
            ci = 2 * i
            pltpu.async_copy(p_hbm.at[src_v.at[ci + 1]], rows_v.at[1], sem1)
            pltpu.make_async_copy(p_hbm.at[src_v.at[ci]], rows_v.at[0],
                                  sem0).wait()
            pltpu.sync_copy(rows_v.at[0], acc.at[dst_v.at[ci]], add=True)

            @pl.when(ci + 2 < _SLAB)
            def _():
                pltpu.async_copy(p_hbm.at[src_v.at[ci + 2]], rows_v.at[0],
                                 sem0)

            pltpu.make_async_copy(p_hbm.at[src_v.at[ci + 1]], rows_v.at[1],
                                  sem1).wait()
            pltpu.sync_copy(rows_v.at[1], acc.at[dst_v.at[ci + 1]], add=True)
            return 0

        lax.fori_loop(0, _SLAB // 2, body, 0)


def _agg_body(p_hbm, src_hbm, dst_hbm, out_hbm, src_v, dst_v, rows_v,
              acc, sem0, sem1):
    c = lax.axis_index("c")
    s = lax.axis_index("s")
    zero16 = jnp.zeros((16,), jnp.float32)

    def fill(r, _):
        for j in range(_D // 16):
            rows_v[0, r, pl.ds(j * 16, 16)] = zero16
        return 0

    lax.fori_loop(0, _K, fill, 0)
    for j in range(_RPT // _K):
        pltpu.sync_copy(rows_v.at[0], acc.at[pl.ds(s * _RPT + j * _K, _K)])
    plsc.subcore_barrier()

    base = (1 - c) * s * _CHA + c * (_NS * _CHA + s * _CHB)

    @pl.when(c == 0)
    def _():
        _agg_pipeline(p_hbm, src_hbm, dst_hbm, src_v, dst_v, rows_v, acc,
                      sem0, sem1, base, _CHA)

    @pl.when(c == 1)
    def _():
        _agg_pipeline(p_hbm, src_hbm, dst_hbm, src_v, dst_v, rows_v, acc,
                      sem0, sem1, base, _CHB)

    plsc.subcore_barrier()
    pltpu.sync_copy(acc.at[pl.ds(s * _RPT, _RPT)],
                    out_hbm.at[c, pl.ds(s * _RPT, _RPT)])


def _sc_agg(p, src_r, dst_r):
    return pl.kernel(
        _agg_body,
        out_type=jax.ShapeDtypeStruct((_NC, _NP, _D), jnp.float32),
        mesh=_get_mesh(),
        scratch_types=[
            pltpu.VMEM((_SLAB, _K), jnp.int32),
            pltpu.VMEM((_SLAB, _K), jnp.int32),
            pltpu.VMEM((2, _K, _D), jnp.float32),
            pltpu.VMEM_SHARED((_NP, _D), jnp.float32),
            pltpu.SemaphoreType.DMA,
            pltpu.SemaphoreType.DMA,
        ],
    )(p, src_r, dst_r)


def _degsum_body(d_ref, o_ref):
    o_ref[...] = lax.rsqrt(d_ref[0] + d_ref[1] + 1.0)


def _tc_degsum(deg2):
    return pl.pallas_call(
        _degsum_body,
        out_shape=jax.ShapeDtypeStruct((_NP, _D), jnp.float32))(deg2)


def _dinv128(dinv_ref):
    return dinv_ref[...]


def _row_mask():
    rows = lax.broadcasted_iota(jnp.int32, (_NP, _D), 0)
    return (rows < _N0).astype(jnp.float32)


def _mm_body(x_ref, w_ref, o_ref):
    o_ref[...] = jnp.dot(x_ref[...], w_ref[...],
                         preferred_element_type=jnp.float32)


def _tc_mm(x, w):
    return pl.pallas_call(
        _mm_body, out_shape=jax.ShapeDtypeStruct((_NP, _D), jnp.float32))(x, w)


def _prescale_body(h_ref, deg_ref, o_ref):
    o_ref[...] = _dinv128(deg_ref) * h_ref[...]


def _tc_prescale(h, deg2):
    return pl.pallas_call(
        _prescale_body,
        out_shape=jax.ShapeDtypeStruct((_NP, _D), jnp.float32))(h, deg2)


def _bn_relu(a_ref, p_ref, deg_ref, g_ref, be_ref):
    dinv = _dinv128(deg_ref)
    z = dinv * (a_ref[0] + a_ref[1] + p_ref[...])
    mask = _row_mask()
    m = jnp.sum(z * mask, axis=0, keepdims=True) * (1.0 / _N0)
    d = (z - m) * mask
    var = jnp.sum(d * d, axis=0, keepdims=True) * (1.0 / _N0)
    zn = (z - m) * lax.rsqrt(var + _EPS) * g_ref[...] + be_ref[...]
    return jnp.maximum(zn, 0.0) * mask, dinv, mask


def _layer_body(a_ref, p_ref, deg_ref, g_ref, be_ref, w_ref, o_ref):
    r, dinv, _ = _bn_relu(a_ref, p_ref, deg_ref, g_ref, be_ref)
    o_ref[...] = dinv * jnp.dot(r, w_ref[...],
                                preferred_element_type=jnp.float32)


def _tc_layer(a, p, deg2, g, be, w):
    return pl.pallas_call(
        _layer_body,
        out_shape=jax.ShapeDtypeStruct((_NP, _D), jnp.float32))(
            a, p, deg2, g, be, w)


def _final_body(a_ref, p_ref, deg_ref, g_ref, be_ref, batch_ref, pf_ref,
                pw1_ref, pb1_ref, pw2_ref, pb2_ref,
                cw1_ref, cb1_ref, cw2_ref, cb2_ref, o_ref):
    r, _, mask = _bn_relu(a_ref, p_ref, deg_ref, g_ref, be_ref)
    bids = lax.broadcasted_iota(jnp.int32, (_B, _NP), 0)
    bvals = lax.broadcast_in_dim(batch_ref[...], (_B, _NP), (0, 1))
    oht = (bids == bvals).astype(jnp.float32)
    pooled = jnp.dot(oht, r, preferred_element_type=jnp.float32)
    cnt = jnp.dot(oht, mask, preferred_element_type=jnp.float32)
    mean = pooled / jnp.maximum(cnt, 1.0)
    pe = jnp.maximum(
        jnp.dot(pf_ref[...], pw1_ref[...],
                preferred_element_type=jnp.float32) + pb1_ref[...], 0.0)
    pe2 = jnp.dot(pe, pw2_ref[...],
                  preferred_element_type=jnp.float32) + pb2_ref[...]
    cat = jnp.concatenate(
        [mean, lax.broadcast_in_dim(pe2, (_B, 64), (0, 1))], axis=1)
    l1 = jnp.maximum(
        jnp.dot(cat, cw1_ref[...],
                preferred_element_type=jnp.float32) + cb1_ref[...], 0.0)
    logits = jnp.dot(l1, cw2_ref[...],
                     preferred_element_type=jnp.float32) + cb2_ref[...]
    o_ref[...] = lax.broadcast_in_dim(logits, (_B, _D), (0, 1))


def _tc_final(a, p, deg2, g, be, batch_pad, pf, pw1, pb1, pw2, pb2,
              cw1, cb1, cw2, cb2):
    return pl.pallas_call(
        _final_body,
        out_shape=jax.ShapeDtypeStruct((_B, _D), jnp.float32))(
            a, p, deg2, g, be, batch_pad, pf, pw1, pb1, pw2, pb2,
            cw1, cb1, cw2, cb2)


def kernel(x, edge_index, batch, pocket_features,
           conv_w0, conv_b0, bn_g0, bn_b0,
           conv_w1, conv_b1, bn_g1, bn_b1,
           conv_w2, conv_b2, bn_g2, bn_b2,
           pm_w1, pm_b1, pm_w2, pm_b2,
           cls_w1, cls_b1, cls_w2, cls_b2):
    pad_e = _EP - _E0
    src = jnp.concatenate(
        [edge_index[0], jnp.full((pad_e,), _N0, jnp.int32)])
    dst = jnp.concatenate(
        [edge_index[1], jnp.full((pad_e,), _NP - 1, jnp.int32)])
    src_r = src.reshape(_NC * _NS * _CH, _K)
    dst_r = dst.reshape(_NC * _NS * _CH, _K)
    x_pad = jnp.pad(x, ((0, _NP - _N0), (0, 0)))
    batch_pad = jnp.concatenate(
        [batch, jnp.full((_NP - _N0,), -1, jnp.int32)]).reshape(1, _NP)
    pf = pocket_features.reshape(1, _PD)

    deg2 = _sc_deg(dst_r)
    dinva = _tc_degsum(deg2)
    h0 = _tc_mm(x_pad, conv_w0)
    p0 = _tc_prescale(h0, dinva)
    a0 = _sc_agg(p0, src_r, dst_r)
    p1 = _tc_layer(a0, p0, dinva, bn_g0, bn_b0, conv_w1)
    a1 = _sc_agg(p1, src_r, dst_r)
    p2 = _tc_layer(a1, p1, dinva, bn_g1, bn_b1, conv_w2)
    a2 = _sc_agg(p2, src_r, dst_r)
    out = _tc_final(a2, p2, dinva, bn_g2, bn_b2, batch_pad, pf,
                    pm_w1, pm_b1, pm_w2, pm_b2,
                    cls_w1, cls_b1, cls_w2, cls_b2)
    return out[:, 0]
